# Initial kernel scaffold; baseline (speedup 1.0000x reference)
import jax
import jax.numpy as jnp
from jax.experimental import pallas as pl


def kernel(x, pos_h, pos_w, table_h, table_w):
    raise NotImplementedError("write your pallas kernel here")


# speedup vs baseline: 22.4495x; 22.4495x over previous
import jax
import jax.numpy as jnp
from jax.experimental import pallas as pl


def _stripe_kernel(pos_h_ref, pos_w_ref, th_ref, tw_ref, x_ref, o_ref):
    n_pos = th_ref.shape[0]
    c = th_ref.shape[1]
    w0 = pos_h_ref.shape[2]
    w = x_ref.shape[3]

    ph = pos_h_ref[0]
    pw = pos_w_ref[0]
    rows = jax.lax.broadcasted_iota(jnp.int32, (n_pos, w0), 0)
    oh_h = (rows == ph).astype(jnp.float32)
    oh_w = (rows == pw).astype(jnp.float32)
    s = jax.lax.dot_general(
        th_ref[...], oh_h, (((0,), (0,)), ((), ())),
        preferred_element_type=jnp.float32,
    ) + jax.lax.dot_general(
        tw_ref[...], oh_w, (((0,), (0,)), ((), ())),
        preferred_element_type=jnp.float32,
    )
    ups = (
        jax.lax.broadcasted_iota(jnp.int32, (w0, w), 0)
        == jax.lax.broadcasted_iota(jnp.int32, (w0, w), 1) // (w // w0)
    ).astype(jnp.float32)
    a = jax.lax.dot_general(
        s, ups, (((1,), (0,)), ((), ())), preferred_element_type=jnp.float32
    )
    o_ref[0] = x_ref[0] + a[:, None, :]


def kernel(x, pos_h, pos_w, table_h, table_w):
    B, C, H, W = x.shape
    H0, W0 = pos_h.shape[1], pos_h.shape[2]
    hb = H // H0
    ph = pos_h.reshape(B * H0, 1, W0)
    pw = pos_w.reshape(B * H0, 1, W0)

    grid = (B * H0,)
    return pl.pallas_call(
        _stripe_kernel,
        grid=grid,
        in_specs=[
            pl.BlockSpec((1, 1, W0), lambda i: (i, 0, 0)),
            pl.BlockSpec((1, 1, W0), lambda i: (i, 0, 0)),
            pl.BlockSpec(table_h.shape, lambda i: (0, 0)),
            pl.BlockSpec(table_w.shape, lambda i: (0, 0)),
            pl.BlockSpec((1, C, hb, W), lambda i: (i // H0, 0, i % H0, 0)),
        ],
        out_specs=pl.BlockSpec((1, C, hb, W), lambda i: (i // H0, 0, i % H0, 0)),
        out_shape=jax.ShapeDtypeStruct(x.shape, x.dtype),
    )(ph, pw, table_h, table_w, x)
